# baseline (speedup 1.0000x reference)
import functools

import jax
import jax.numpy as jnp
from jax.experimental import pallas as pl

_EPS_BN = 1e-5
_EPS_DEN = 1e-6


def _mm_kernel(x_ref, w_ref, b_ref, o_ref):
    o_ref[...] = (
        jnp.dot(x_ref[...], w_ref[...], preferred_element_type=jnp.float32)
        + b_ref[...]
    )


def _mm(x, w, b, bn):
    n, k = x.shape
    m = w.shape[1]
    return pl.pallas_call(
        _mm_kernel,
        grid=(n // bn,),
        in_specs=[
            pl.BlockSpec((bn, k), lambda i: (i, 0)),
            pl.BlockSpec((k, m), lambda i: (0, 0)),
            pl.BlockSpec((1, m), lambda i: (0, 0)),
        ],
        out_specs=pl.BlockSpec((bn, m), lambda i: (i, 0)),
        out_shape=jax.ShapeDtypeStruct((n, m), jnp.float32),
    )(x, w, b.reshape(1, m))


def _node_update_kernel(ax_ref, num_ref, den_ref, xin_ref, g_ref, b_ref, o_ref):
    hn = ax_ref[...] + num_ref[...] / (den_ref[...] + _EPS_DEN)
    hn = g_ref[...] * hn * (1.0 / (1.0 + _EPS_BN) ** 0.5) + b_ref[...]
    o_ref[...] = xin_ref[...] + jnp.maximum(hn, 0.0)


def _node_update(ax, num, den, x_in, g, b, bn):
    n, m = ax.shape
    return pl.pallas_call(
        _node_update_kernel,
        grid=(n // bn,),
        in_specs=[
            pl.BlockSpec((bn, m), lambda i: (i, 0)),
            pl.BlockSpec((bn, m), lambda i: (i, 0)),
            pl.BlockSpec((bn, m), lambda i: (i, 0)),
            pl.BlockSpec((bn, m), lambda i: (i, 0)),
            pl.BlockSpec((1, m), lambda i: (0, 0)),
            pl.BlockSpec((1, m), lambda i: (0, 0)),
        ],
        out_specs=pl.BlockSpec((bn, m), lambda i: (i, 0)),
        out_shape=jax.ShapeDtypeStruct((n, m), jnp.float32),
    )(ax, num, den, x_in, g.reshape(1, m), b.reshape(1, m))


def _edge_update_kernel(eij_ref, ein_ref, g_ref, b_ref, o_ref):
    en = g_ref[...] * eij_ref[...] * (1.0 / (1.0 + _EPS_BN) ** 0.5) + b_ref[...]
    o_ref[...] = ein_ref[...] + jnp.maximum(en, 0.0)


def _edge_update(eij, e_in, g, b, bn):
    n, m = eij.shape
    return pl.pallas_call(
        _edge_update_kernel,
        grid=(n // bn,),
        in_specs=[
            pl.BlockSpec((bn, m), lambda i: (i, 0)),
            pl.BlockSpec((bn, m), lambda i: (i, 0)),
            pl.BlockSpec((1, m), lambda i: (0, 0)),
            pl.BlockSpec((1, m), lambda i: (0, 0)),
        ],
        out_specs=pl.BlockSpec((bn, m), lambda i: (i, 0)),
        out_shape=jax.ShapeDtypeStruct((n, m), jnp.float32),
    )(eij, e_in, g.reshape(1, m), b.reshape(1, m))


def kernel(x, edge_attr, edge_index, node_W, node_b, edge_W, edge_b,
           A_W, A_b, B_W, B_b, C_W, C_b, D_W, D_b, E_W, E_b,
           bnx_g, bnx_b, bne_g, bne_b):
    src = edge_index[0]
    dst = edge_index[1]
    n = x.shape[0]
    H = node_W.shape[1]
    L = A_W.shape[0]

    h = _mm(x, node_W, node_b, 2000)
    e = _mm(edge_attr, edge_W, edge_b, 2000)

    for l in range(L):
        x_in, e_in = h, e
        W_cat = jnp.concatenate([A_W[l], B_W[l], D_W[l], E_W[l]], axis=1)
        b_cat = jnp.concatenate([A_b[l], B_b[l], D_b[l], E_b[l]], axis=0)
        proj = _mm(h, W_cat, b_cat, 2000)
        Ax = proj[:, 0 * H:1 * H]
        Bx = proj[:, 1 * H:2 * H]
        Dx = proj[:, 2 * H:3 * H]
        Ex = proj[:, 3 * H:4 * H]
        Ce = _mm(e, C_W[l], C_b[l], 2000)

        e_ij = Dx[dst] + Ex[src] + Ce
        sigma = jax.nn.sigmoid(e_ij)
        num = jax.ops.segment_sum(sigma * Bx[src], dst, num_segments=n)
        den = jax.ops.segment_sum(sigma, dst, num_segments=n)

        h = _node_update(Ax, num, den, x_in, bnx_g[l], bnx_b[l], 2000)
        e = _edge_update(e_ij, e_in, bne_g[l], bne_b[l], 2000)
    return (h, e)
